# baseline (speedup 1.0000x reference)
import functools

import jax
import jax.numpy as jnp
from jax import lax
from jax.experimental import pallas as pl
from jax.experimental.pallas import tpu as pltpu
from jax.experimental.pallas import tpu_sc as plsc

B = 16384
EMB = 32
NC, NS = 2, 16
NW = NC * NS
BPW = B // NW
CHUNK = 128
NCH = BPW // CHUNK

@functools.lru_cache(maxsize=None)
def _make_sc_gather():
  mesh = plsc.VectorSubcoreMesh(
      core_axis_name="c", subcore_axis_name="s", num_cores=NC, num_subcores=NS)

  @functools.partial(
      pl.kernel,
      out_type=[jax.ShapeDtypeStruct((B, EMB), jnp.float32)] * 4,
      mesh=mesh,
      scratch_types=[
          pltpu.VMEM((NCH, CHUNK), jnp.int32),
          pltpu.VMEM((NCH, CHUNK), jnp.int32),
          pltpu.VMEM((BPW, EMB), jnp.float32),
          pltpu.VMEM((BPW, EMB), jnp.float32),
          pltpu.VMEM((BPW, EMB), jnp.float32),
          pltpu.VMEM((BPW, EMB), jnp.float32),
          pltpu.SemaphoreType.DMA,
          pltpu.SemaphoreType.DMA,
          pltpu.SemaphoreType.DMA,
          pltpu.SemaphoreType.DMA,
          pltpu.SemaphoreType.DMA,
      ],
      compiler_params=pltpu.CompilerParams(use_tc_tiling_on_sc=False),
  )
  def sc_gather(user_h, item_h, ue_mlp_h, ie_mlp_h, ue_gmf_h, ie_gmf_h,
                out_um_h, out_im_h, out_ug_h, out_ig_h,
                uidx_v, iidx_v, um_v, im_v, ug_v, ig_v,
                sem0, sem1, sem2, sem3, wsem):
    wid = lax.axis_index("s") * NC + lax.axis_index("c")
    base = wid * BPW
    for j in range(NCH):
        pltpu.sync_copy(user_h.at[pl.ds(base + j * CHUNK, CHUNK)], uidx_v.at[j])
        pltpu.sync_copy(item_h.at[pl.ds(base + j * CHUNK, CHUNK)], iidx_v.at[j])

    tables = ((ue_mlp_h, uidx_v, um_v, out_um_h, sem0),
              (ie_mlp_h, iidx_v, im_v, out_im_h, sem1),
              (ue_gmf_h, uidx_v, ug_v, out_ug_h, sem2),
              (ie_gmf_h, iidx_v, ig_v, out_ig_h, sem3))

    copies = []
    for table_h, idx_v, rows_v, _, sem in tables:
        for j in range(NCH):
            copies.append(pltpu.async_copy(
                table_h.at[idx_v.at[j]],
                rows_v.at[pl.ds(j * CHUNK, CHUNK)],
                sem))
    k = 0
    wcopies = []
    for table_h, idx_v, rows_v, out_h, sem in tables:
        for j in range(NCH):
            copies[k].wait()
            k += 1
        wcopies.append(pltpu.async_copy(rows_v, out_h.at[pl.ds(base, BPW)], wsem))
    for wc in wcopies:
        wc.wait()

  return sc_gather


BLK = 2048


def _tc_body(um_ref, im_ref, ug_ref, ig_ref,
             w1_ref, b1_ref, w2_ref, b2_ref, wo_ref, bo_ref, out_ref):
    h = jnp.dot(um_ref[...], w1_ref[0:EMB, :], preferred_element_type=jnp.float32)
    h = h + jnp.dot(im_ref[...], w1_ref[EMB:, :], preferred_element_type=jnp.float32)
    h = jnp.maximum(h + b1_ref[...], 0.0)
    m = jnp.dot(h, w2_ref[...], preferred_element_type=jnp.float32)
    m = jnp.maximum(m + b2_ref[...], 0.0)
    g = ug_ref[...] * ig_ref[...]
    o = jnp.dot(g, wo_ref[0:EMB, :], preferred_element_type=jnp.float32)
    o = o + jnp.dot(m, wo_ref[EMB:, :], preferred_element_type=jnp.float32)
    out_ref[...] = (o + bo_ref[...])[:, 0]


def _tc_dense(um, im, ug, ig, W1, b1, W2, b2, Wo, bo):
    grid = (B // BLK,)
    row_spec = pl.BlockSpec((BLK, EMB), lambda i: (i, 0))
    full = lambda shape: pl.BlockSpec(shape, lambda i: (0,) * len(shape))
    return pl.pallas_call(
        _tc_body,
        grid=grid,
        in_specs=[row_spec, row_spec, row_spec, row_spec,
                  full((2 * EMB, 64)), full((1, 64)),
                  full((64, EMB)), full((1, EMB)),
                  full((2 * EMB, 1)), full((1, 1))],
        out_specs=pl.BlockSpec((BLK,), lambda i: (i,)),
        out_shape=jax.ShapeDtypeStruct((B,), jnp.float32),
        compiler_params=pltpu.CompilerParams(
            dimension_semantics=("arbitrary",)),
    )(um, im, ug, ig, W1, b1, W2, b2, Wo, bo)


def kernel(user, item, user_emb_mlp, item_emb_mlp, user_emb_gmf, item_emb_gmf,
           W1, b1, W2, b2, Wo, bo):
    user = user.astype(jnp.int32)
    item = item.astype(jnp.int32)
    um, im, ug, ig = _make_sc_gather()(user, item, user_emb_mlp, item_emb_mlp,
                                       user_emb_gmf, item_emb_gmf)
    return _tc_dense(um, im, ug, ig,
                     W1, b1.reshape(1, -1), W2, b2.reshape(1, -1),
                     Wo, bo.reshape(1, 1))
